# baseline (speedup 1.0000x reference)
import functools

import jax
import jax.numpy as jnp
from jax import lax
from jax.experimental import pallas as pl
from jax.experimental.pallas import tpu as pltpu
from jax.experimental.pallas import tpu_sc as plsc

N_NODES = 10000
K = 32
D_FEAT = 128
B = N_NODES * K
NW = 32
BPW = B // NW
CHUNK = 80
NCHUNK = BPW // CHUNK
GPS = 5
SUPER = CHUNK * GPS
NSUPER = BPW // SUPER
LANES = 16


def _gather_rows(table, idx):
    mesh = plsc.VectorSubcoreMesh(core_axis_name="c", subcore_axis_name="s")

    @functools.partial(
        pl.kernel,
        mesh=mesh,
        out_type=jax.ShapeDtypeStruct((B, D_FEAT), jnp.float32),
        scratch_types=[
            pltpu.VMEM((BPW,), jnp.int32),
            pltpu.VMEM((SUPER, D_FEAT), jnp.float32),
            pltpu.VMEM((SUPER, D_FEAT), jnp.float32),
            pltpu.SemaphoreType.DMA,
            pltpu.SemaphoreType.DMA,
            pltpu.SemaphoreType.DMA,
            pltpu.SemaphoreType.DMA,
        ],
    )
    def k(table_hbm, idx_hbm, out_hbm, idx_v, bufa, bufb, gsa, gsb, wsa, wsb):
        nc = 2
        wid = lax.axis_index("s") * nc + lax.axis_index("c")
        base = pl.multiple_of(wid * BPW, 8)

        pltpu.sync_copy(idx_hbm.at[pl.ds(base, BPW)], idx_v)

        def fix_super(s):
            off0 = s * SUPER
            for i in range(SUPER // LANES):
                o = pl.multiple_of(off0 + i * LANES, 8)
                v = idx_v[pl.ds(o, LANES)]
                idx_v[pl.ds(o, LANES)] = jnp.where(v < 0, N_NODES, v)

        def fire_gathers(s, buf, gsem):
            for c in range(GPS):
                off = pl.multiple_of(s * SUPER + c * CHUNK, 8)
                pltpu.async_copy(
                    table_hbm.at[idx_v.at[pl.ds(off, CHUNK)]],
                    buf.at[pl.ds(c * CHUNK, CHUNK)],
                    gsem,
                )

        def drain_gathers(s, buf, gsem):
            for c in range(GPS):
                off = pl.multiple_of(s * SUPER + c * CHUNK, 8)
                pltpu.make_async_copy(
                    table_hbm.at[idx_v.at[pl.ds(off, CHUNK)]],
                    buf.at[pl.ds(c * CHUNK, CHUNK)],
                    gsem,
                ).wait()

        def slot(s, buf, gsem, wsem):
            @pl.when(s >= 2)
            def _():
                poff = pl.multiple_of((s - 2) * SUPER, 8)
                pltpu.make_async_copy(
                    buf, out_hbm.at[pl.ds(base + poff, SUPER)], wsem
                ).wait()

            fix_super(s)
            fire_gathers(s, buf, gsem)
            drain_gathers(s, buf, gsem)
            off = pl.multiple_of(s * SUPER, 8)
            pltpu.async_copy(buf, out_hbm.at[pl.ds(base + off, SUPER)], wsem)

        def round_(g, carry):
            slot(g * 2, bufa, gsa, wsa)

            @pl.when(g * 2 + 1 < NSUPER)
            def _():
                slot(g * 2 + 1, bufb, gsb, wsb)

            return carry

        lax.fori_loop(0, (NSUPER + 1) // 2, round_, 0)

        pltpu.make_async_copy(
            bufa, out_hbm.at[pl.ds(base + (NSUPER - 1) * SUPER, SUPER)], wsa
        ).wait()
        pltpu.make_async_copy(
            bufb, out_hbm.at[pl.ds(base + (NSUPER - 2) * SUPER, SUPER)], wsb
        ).wait()

    return k(table, idx)


def kernel(features, nidx):
    table = jnp.concatenate(
        [features, jnp.zeros((1, D_FEAT), jnp.float32)], axis=0
    )
    idx = nidx.astype(jnp.int32).reshape(B)
    out = _gather_rows(table, idx)
    return out.reshape(N_NODES, K * D_FEAT)


# speedup vs baseline: 1.2176x; 1.2176x over previous
import functools

import jax
import jax.numpy as jnp
from jax import lax
from jax.experimental import pallas as pl
from jax.experimental.pallas import tpu as pltpu
from jax.experimental.pallas import tpu_sc as plsc

N_NODES = 10000
K = 32
D_FEAT = 128
B = N_NODES * K
NW = 32
BPW = B // NW
CHUNK = 80
NCHUNK = BPW // CHUNK
RING = 3
LANES = 16
NSUB = 16
T_ROWS = 10112
T_PER_SUB = T_ROWS // NSUB


def _gather_rows(table, idx):
    mesh = plsc.VectorSubcoreMesh(core_axis_name="c", subcore_axis_name="s")

    @functools.partial(
        pl.kernel,
        mesh=mesh,
        out_type=jax.ShapeDtypeStruct((B, D_FEAT), jnp.float32),
        scratch_types=[
            pltpu.VMEM((BPW,), jnp.int32),
        ]
        + [pltpu.VMEM((CHUNK, D_FEAT), jnp.float32) for _ in range(RING)]
        + [pltpu.SemaphoreType.DMA for _ in range(RING)]
        + [pltpu.VMEM_SHARED((T_ROWS, D_FEAT), jnp.float32)],
    )
    def k(table_hbm, idx_hbm, out_hbm, idx_v, *rest):
        bufs = rest[:RING]
        sems = rest[RING:2 * RING]
        shared = rest[2 * RING]
        nc = 2
        sid = lax.axis_index("s")
        wid = sid * nc + lax.axis_index("c")
        base = pl.multiple_of(wid * BPW, 8)

        soff = pl.multiple_of(sid * T_PER_SUB, 8)
        pltpu.sync_copy(
            table_hbm.at[pl.ds(soff, T_PER_SUB)],
            shared.at[pl.ds(soff, T_PER_SUB)],
        )

        pltpu.sync_copy(idx_hbm.at[pl.ds(base, BPW)], idx_v)
        plsc.subcore_barrier()

        def fix_chunk(off):
            for i in range(CHUNK // LANES):
                o = pl.multiple_of(off + i * LANES, 8)
                v = idx_v[pl.ds(o, LANES)]
                idx_v[pl.ds(o, LANES)] = jnp.where(v < 0, N_NODES, v)

        def fire(off, b):
            pltpu.async_copy(
                shared.at[idx_v.at[pl.ds(off, CHUNK)]], bufs[b], sems[b]
            )

        for b in range(RING):
            fix_chunk(b * CHUNK)
            fire(b * CHUNK, b)

        def round_(g, carry):
            for b in range(RING):
                j = g * RING + b
                off = pl.multiple_of(j * CHUNK, 8)

                @pl.when(j < NCHUNK)
                def _():
                    pltpu.make_async_copy(
                        shared.at[idx_v.at[pl.ds(off, CHUNK)]],
                        bufs[b],
                        sems[b],
                    ).wait()
                    pltpu.sync_copy(
                        bufs[b], out_hbm.at[pl.ds(base + off, CHUNK)]
                    )

                    @pl.when(j + RING < NCHUNK)
                    def _():
                        noff = pl.multiple_of((j + RING) * CHUNK, 8)
                        fix_chunk(noff)
                        fire(noff, b)

            return carry

        lax.fori_loop(0, (NCHUNK + RING - 1) // RING, round_, 0)

    return k(table, idx)


def kernel(features, nidx):
    table = jnp.concatenate(
        [features, jnp.zeros((T_ROWS - N_NODES, D_FEAT), jnp.float32)], axis=0
    )
    idx = nidx.astype(jnp.int32).reshape(B)
    out = _gather_rows(table, idx)
    return out.reshape(N_NODES, K * D_FEAT)
